# Initial kernel scaffold; baseline (speedup 1.0000x reference)
import jax
import jax.numpy as jnp
from jax.experimental import pallas as pl


def kernel(x, edge_index, edges, rel_embed, Wq, bq, Wk, Wv, Wo, bo, ln1_g, ln1_b, W1, b1, W2, b2, ln2_g, ln2_b):
    raise NotImplementedError("write your pallas kernel here")


# speedup vs baseline: 31.7495x; 31.7495x over previous
import functools

import jax
import jax.numpy as jnp
import numpy as np
from jax import lax
from jax.experimental import pallas as pl
from jax.experimental.pallas import tpu as pltpu
from jax.experimental.pallas import tpu_sc as plsc

N = 10000
E = 320000
D = 128
H = 8
DK = 16
R = 100
DFF = 4 * D
NC, NS = 2, 16
NW = NC * NS
EPT = E // NW
CH = 40
NCHUNK = EPT // CH
NPAD = 10000
ACCR = 11264
ZROWS = ACCR - NPAD
RPT = ACCR // NS

BN = 1000
_DN = (((1,), (1,)), ((), ()))


def _ln(t, g, b):
    m = jnp.mean(t, axis=-1, keepdims=True)
    v = jnp.mean((t - m) ** 2, axis=-1, keepdims=True)
    return (t - m) / jnp.sqrt(v + 1e-5) * g + b


def _front_body(x_ref, wq_ref, bq_ref, wk_ref, wv_ref, q_ref, k_ref, v_ref):
    x = x_ref[...]
    f32 = jnp.float32
    q_ref[...] = lax.dot_general(x, wq_ref[...], _DN, preferred_element_type=f32) + bq_ref[...]
    k_ref[...] = lax.dot_general(x, wk_ref[...], _DN, preferred_element_type=f32)
    v_ref[...] = lax.dot_general(x, wv_ref[...], _DN, preferred_element_type=f32)


def _tc_front(x, Wq, bq, Wk, Wv):
    rows = pl.BlockSpec((BN, D), lambda i: (i, 0))
    wsp = pl.BlockSpec((D, D), lambda i: (0, 0))
    bsp = pl.BlockSpec((1, D), lambda i: (0, 0))
    return pl.pallas_call(
        _front_body,
        grid=(N // BN,),
        in_specs=[rows, wsp, bsp, wsp, wsp],
        out_specs=[rows, rows, rows],
        out_shape=[jax.ShapeDtypeStruct((N, D), jnp.float32)] * 3,
    )(x, Wq, bq.reshape(1, D), Wk, Wv)


def _post_body(with_qkv, x_ref, p0_ref, p1_ref, z_ref, wo_ref, bo_ref, g1_ref, be1_ref,
               w1_ref, b1_ref, w2_ref, b2_ref, g2_ref, be2_ref, *rest):
    if with_qkv:
        wqn_ref, bqn_ref, wkn_ref, wvn_ref, xo_ref, qo_ref, ko_ref, vo_ref = rest
    else:
        (xo_ref,) = rest
    f32 = jnp.float32
    wv = p0_ref[...] + p1_ref[...]
    z = jnp.sum(z_ref[...], axis=0)
    hrow = lax.broadcasted_iota(jnp.int32, (H, D), 0)
    hcol = lax.broadcasted_iota(jnp.int32, (H, D), 1) // DK
    bmat = (hrow == hcol).astype(f32)
    zexp = lax.dot_general(z, bmat, (((1,), (0,)), ((), ())), preferred_element_type=f32)
    o = wv / zexp
    t = x_ref[...] + lax.dot_general(o, wo_ref[...], _DN, preferred_element_type=f32) + bo_ref[...]
    h = _ln(t, g1_ref[...], be1_ref[...])
    ff = jnp.maximum(lax.dot_general(h, w1_ref[...], _DN, preferred_element_type=f32) + b1_ref[...], 0.0)
    ff2 = lax.dot_general(ff, w2_ref[...], _DN, preferred_element_type=f32) + b2_ref[...]
    xn = _ln(h + ff2, g2_ref[...], be2_ref[...])
    xo_ref[...] = xn
    if with_qkv:
        qo_ref[...] = lax.dot_general(xn, wqn_ref[...], _DN, preferred_element_type=f32) + bqn_ref[...]
        ko_ref[...] = lax.dot_general(xn, wkn_ref[...], _DN, preferred_element_type=f32)
        vo_ref[...] = lax.dot_general(xn, wvn_ref[...], _DN, preferred_element_type=f32)


def _tc_post(x, p0, p1, zp, Wo, bo, g1, be1, W1, b1, W2, b2, g2, be2, nxt=None):
    rows = pl.BlockSpec((BN, D), lambda i: (i, 0))
    zsp = pl.BlockSpec((NC, BN, H), lambda i: (0, i, 0))
    wsp = pl.BlockSpec((D, D), lambda i: (0, 0))
    bsp = pl.BlockSpec((1, D), lambda i: (0, 0))
    w1sp = pl.BlockSpec((DFF, D), lambda i: (0, 0))
    b1sp = pl.BlockSpec((1, DFF), lambda i: (0, 0))
    w2sp = pl.BlockSpec((D, DFF), lambda i: (0, 0))
    in_specs = [rows, rows, rows, zsp, wsp, bsp, bsp, bsp, w1sp, b1sp, w2sp, bsp, bsp, bsp]
    args = [x, p0, p1, zp, Wo, bo.reshape(1, D), g1.reshape(1, D), be1.reshape(1, D),
            W1, b1.reshape(1, DFF), W2, b2.reshape(1, D), g2.reshape(1, D), be2.reshape(1, D)]
    with_qkv = nxt is not None
    nout = 4 if with_qkv else 1
    if with_qkv:
        Wqn, bqn, Wkn, Wvn = nxt
        in_specs += [wsp, bsp, wsp, wsp]
        args += [Wqn, bqn.reshape(1, D), Wkn, Wvn]
    return pl.pallas_call(
        functools.partial(_post_body, with_qkv),
        grid=(N // BN,),
        in_specs=in_specs,
        out_specs=[rows] * nout,
        out_shape=[jax.ShapeDtypeStruct((N, D), jnp.float32)] * nout,
    )(*args)


def _make_perms(lane):
    def c2(v):
        return v.reshape(16, 1)
    return {
        "rot8": c2((lane + 8) & 15),
        "w8r4": c2((lane & 8) | ((lane + 4) & 7)),
        "l2": c2((lane & 3) | ((lane & 4) << 1)),
        "w4r2": c2((lane & 12) | ((lane + 2) & 3)),
        "l3": c2((((lane & 7) >> 1) << 2) | (lane & 1)),
        "w2r1": c2(lane ^ 1),
        "cmp": c2((lane & 7) << 1),
    }


_GDN = lax.GatherDimensionNumbers(
    offset_dims=(), collapsed_slice_dims=(0,), start_index_map=(0,))


def _perm(x, idx):
    return lax.gather(x, idx, _GDN, (1,),
                      mode=lax.GatherScatterMode.PROMISE_IN_BOUNDS)


def _sum8(ts, m8, P):
    a = []
    for i in range(4):
        lo = jnp.where(m8, ts[2 * i], _perm(ts[2 * i + 1], P["rot8"]))
        hi = jnp.where(m8, _perm(ts[2 * i], P["rot8"]), ts[2 * i + 1])
        a.append(lo + hi)
    c = []
    for j in range(2):
        f0 = a[2 * j] + _perm(a[2 * j], P["w8r4"])
        f1 = a[2 * j + 1] + _perm(a[2 * j + 1], P["w8r4"])
        c.append(jnp.where(m8, _perm(f0, P["l2"]), _perm(f1, P["l2"])))
    f0 = c[0] + _perm(c[0], P["w4r2"])
    f1 = c[1] + _perm(c[1], P["w4r2"])
    d = jnp.where(m8, _perm(f0, P["l3"]), _perm(f1, P["l3"]))
    s = d + _perm(d, P["w2r1"])
    return _perm(s, P["cmp"])

def _edge_body(q_hbm, k_hbm, v_hbm, src_hbm, dst_hbm, rel_hbm, re_hbm,
               acc_hbm,
               acc_sh, src_v, dst_v, dstg_v, dst_e, rel_e, re_v, k_v, q_v, v_v,
               out_v, out_z,
               sem_k, sem_q, sem_v):
    c = lax.axis_index("c")
    s = lax.axis_index("s")
    wid = c * NS + s
    f32 = jnp.float32
    zero16 = jnp.zeros((16,), f32)

    def _zrow(r, carry):
        for j in range(D // 16):
            out_v[r, pl.ds(j * 16, 16)] = zero16
            out_z[r, pl.ds(j * 16, 16)] = zero16
        return carry
    lax.fori_loop(0, CH, _zrow, 0)
    for t in range(RPT // CH):
        pltpu.sync_copy(out_v, acc_sh.at[pl.ds(s * RPT + t * CH, CH)])
    _rem = RPT - (RPT // CH) * CH
    if _rem:
        pltpu.sync_copy(out_v.at[pl.ds(0, _rem)],
                        acc_sh.at[pl.ds(s * RPT + (RPT // CH) * CH, _rem)])
    plsc.subcore_barrier()

    lane = lax.iota(jnp.int32, 16)
    m8 = lane < H
    P = _make_perms(lane)
    hb_idx = [jnp.full((16, 1), h, jnp.int32) for h in range(H)]
    pltpu.sync_copy(re_hbm, re_v)

    def _chunk(ci, carry):
        base = wid * EPT + ci * CH
        pltpu.sync_copy(src_hbm.at[pl.ds(base, CH)], src_v)
        pltpu.sync_copy(dst_hbm.at[pl.ds(base, CH)], dst_v)
        pltpu.sync_copy(dst_hbm.at[pl.ds(base, CH)], dst_e.at[pl.ds(0, CH)])
        pltpu.sync_copy(rel_hbm.at[pl.ds(base, CH)], rel_e.at[pl.ds(0, CH)])
        for i in range(CH // 16):
            dv = dst_e[pl.ds(i * 16, 16)]
            dstg_v[pl.ds(i * 16, 16)] = NPAD + (dv >> 3)
        dv = dst_e[pl.ds(CH - 16, 16)]
        dstg_v[pl.ds(CH - 16, 16)] = NPAD + (dv >> 3)
        cpk = pltpu.async_copy(k_hbm.at[src_v], k_v, sem_k)
        cpq = pltpu.async_copy(q_hbm.at[dst_v], q_v, sem_q)
        cpv = pltpu.async_copy(v_hbm.at[src_v], v_v, sem_v)
        cpk.wait()
        cpq.wait()
        cpv.wait()

        def _group(g, gcarry):
            rel_row = rel_e[pl.ds(g * 8, 16)]
            dst_row = dst_e[pl.ds(g * 8, 16)]
            for j in range(8):
                e = g * 8 + j
                rid = rel_row[j]
                did = dst_row[j]
                lgx = re_v[rid, :]
                ts = []
                for h in range(H):
                    kc = k_v[e, pl.ds(h * DK, DK)]
                    qc = q_v[e, pl.ds(h * DK, DK)]
                    ts.append((kc + lgx) * qc)
                svec = _sum8(ts, m8, P)
                svec = jnp.exp(jnp.clip(svec * 0.25, -10.0, 10.0))
                for jz in range(D // 16):
                    out_z[e, pl.ds(jz * 16, 16)] = zero16
                out_z[e, pl.ds((did & 7) * 16, 16)] = jnp.where(m8, svec, 0.0)
                for h in range(H):
                    wb = _perm(svec, hb_idx[h])
                    vc = v_v[e, pl.ds(h * DK, DK)]
                    out_v[e, pl.ds(h * DK, DK)] = (vc + lgx) * wb
            return gcarry
        lax.fori_loop(0, CH // 8, _group, 0)
        pltpu.sync_copy(out_v, acc_sh.at[dst_v], add=True)
        pltpu.sync_copy(out_z, acc_sh.at[dstg_v], add=True)
        return carry
    lax.fori_loop(0, NCHUNK, _chunk, 0)

    plsc.subcore_barrier()
    pltpu.sync_copy(acc_sh.at[pl.ds(s * RPT, RPT)],
                    acc_hbm.at[c, pl.ds(s * RPT, RPT)])


def _sc_edge(q, k, v, src, dst, rel, rel_embed):
    f32 = jnp.float32
    mesh = plsc.VectorSubcoreMesh(core_axis_name="c", subcore_axis_name="s")
    fn = pl.kernel(
        _edge_body,
        out_type=jax.ShapeDtypeStruct((NC, ACCR, D), f32),
        mesh=mesh,
        scratch_types=[
            pltpu.VMEM_SHARED((ACCR, D), f32),
            pltpu.VMEM((CH,), jnp.int32),
            pltpu.VMEM((CH,), jnp.int32),
            pltpu.VMEM((CH,), jnp.int32),
            pltpu.VMEM((CH + 16,), jnp.int32),
            pltpu.VMEM((CH + 16,), jnp.int32),
            pltpu.VMEM((R, DK), f32),
            pltpu.VMEM((CH, D), f32),
            pltpu.VMEM((CH, D), f32),
            pltpu.VMEM((CH, D), f32),
            pltpu.VMEM((CH, D), f32),
            pltpu.VMEM((CH, D), f32),
            pltpu.SemaphoreType.DMA,
            pltpu.SemaphoreType.DMA,
            pltpu.SemaphoreType.DMA,
        ],
    )
    return fn(q, k, v, src, dst, rel, rel_embed)


def kernel(x, edge_index, edges, rel_embed, Wq, bq, Wk, Wv, Wo, bo,
           ln1_g, ln1_b, W1, b1, W2, b2, ln2_g, ln2_b):
    src = edge_index[0]
    dst = edge_index[1]
    def _split(acc):
        wv0 = acc[0, :N]
        wv1 = acc[1, :N]
        zp = acc[:, NPAD:].reshape(NC, ZROWS * 8, 16)[:, :N, :H]
        return wv0, wv1, zp

    q0, k0, v0 = _tc_front(x, Wq[0], bq[0], Wk[0], Wv[0])
    wv0, wv1, zp = _split(_sc_edge(q0, k0, v0, src, dst, edges, rel_embed))
    x1, q1, k1, v1 = _tc_post(x, wv0, wv1, zp, Wo[0], bo[0],
                              ln1_g[0], ln1_b[0], W1[0], b1[0], W2[0], b2[0],
                              ln2_g[0], ln2_b[0],
                              nxt=(Wq[1], bq[1], Wk[1], Wv[1]))
    wv0, wv1, zp = _split(_sc_edge(q1, k1, v1, src, dst, edges, rel_embed))
    (x2,) = _tc_post(x1, wv0, wv1, zp, Wo[1], bo[1],
                     ln1_g[1], ln1_b[1], W1[1], b1[1], W2[1], b2[1],
                     ln2_g[1], ln2_b[1])
    return x2
